# baseline (speedup 1.0000x reference)
import functools

import jax
import jax.numpy as jnp
from jax import lax
from jax.experimental import pallas as pl
from jax.experimental.pallas import tpu as pltpu
from jax.experimental.pallas import tpu_sc as plsc

_D = 1024
_V = 11
_NW = 32
_CH = 16
_LANES = 16


def _make_sc_kernel(n_rows):
    rows_w = n_rows // _NW
    nch = rows_w // _CH
    mesh = plsc.VectorSubcoreMesh(core_axis_name="c", subcore_axis_name="s")
    buf = pltpu.VMEM((_CH, _D), jnp.float32)

    @functools.partial(
        pl.kernel,
        mesh=mesh,
        out_type=jax.ShapeDtypeStruct((n_rows, _D), jnp.float32),
        scratch_types=[
            pltpu.VMEM((rows_w,), jnp.int32),
            buf, buf,
            buf, buf,
            buf, buf,
            pltpu.SemaphoreType.DMA, pltpu.SemaphoreType.DMA,
            pltpu.SemaphoreType.DMA, pltpu.SemaphoreType.DMA,
            pltpu.SemaphoreType.DMA, pltpu.SemaphoreType.DMA,
        ],
    )
    def k(x_hbm, s_hbm, t_hbm, out_hbm, idx_v,
          xb0, xb1, eb0, eb1, sb0, sb1,
          ls0, ls1, gs0, gs1, ss0, ss1):
        wid = lax.axis_index("s") * 2 + lax.axis_index("c")
        base = wid * rows_w
        pltpu.sync_copy(s_hbm.at[pl.ds(base, rows_w)], idx_v)
        xbs, ebs, sbs = (xb0, xb1), (eb0, eb1), (sb0, sb1)
        lss, gss, sss = (ls0, ls1), (gs0, gs1), (ss0, ss1)

        def issue(i, b):
            r0 = base + i * _CH
            pltpu.async_copy(x_hbm.at[pl.ds(r0, _CH)], xbs[b], lss[b])
            pltpu.async_copy(t_hbm.at[idx_v.at[pl.ds(i * _CH, _CH)]],
                             ebs[b], gss[b])

        issue(0, 0)
        issue(1, 1)

        def pair(i2, carry):
            for b in (0, 1):
                i = i2 * 2 + b
                pltpu.make_async_copy(x_hbm.at[pl.ds(0, _CH)],
                                      xbs[b], lss[b]).wait()
                pltpu.make_async_copy(x_hbm.at[pl.ds(0, _CH)],
                                      ebs[b], gss[b]).wait()
                @pl.when(i2 > 0)
                def _():
                    pltpu.make_async_copy(sbs[b], out_hbm.at[pl.ds(0, _CH)],
                                          sss[b]).wait()
                xb, eb, sb = xbs[b], ebs[b], sbs[b]
                pltpu.async_copy(xb, out_hbm.at[pl.ds(base + i * _CH, _CH)],
                                 sss[b])
                @pl.when(i + 2 < nch)
                def _():
                    issue(i + 2, b)
            return carry

        lax.fori_loop(0, nch // 2, pair, 0)
        for b in (0, 1):
            pltpu.make_async_copy(sbs[b], out_hbm.at[pl.ds(0, _CH)],
                                  sss[b]).wait()

    return k


def kernel(x, scores, score_embeddings):
    B, L, D = x.shape
    n = B * L
    xf = x.reshape(n, D)
    sf = scores.reshape(n).astype(jnp.int32)
    out = _make_sc_kernel(n)(xf, sf, score_embeddings)
    return out.reshape(B, L, D)


# speedup vs baseline: 1.5330x; 1.5330x over previous
import functools

import jax
import jax.numpy as jnp
from jax import lax
from jax.experimental import pallas as pl
from jax.experimental.pallas import tpu as pltpu
from jax.experimental.pallas import tpu_sc as plsc

_D = 1024
_V = 11
_NW = 32
_CH = 16
_LANES = 16
_NBUF = 4
_NEB = 2


def _make_sc_kernel(n_rows):
    rows_w = n_rows // _NW
    nch = rows_w // _CH
    mesh = plsc.VectorSubcoreMesh(core_axis_name="c", subcore_axis_name="s")
    buf = pltpu.VMEM((_CH, _D), jnp.float32)
    sem = pltpu.SemaphoreType.DMA

    @functools.partial(
        pl.kernel,
        mesh=mesh,
        out_type=jax.ShapeDtypeStruct((n_rows, _D), jnp.float32),
        scratch_types=(
            [pltpu.VMEM((rows_w,), jnp.int32)]
            + [buf] * _NBUF + [buf] * _NEB
            + [sem] * _NBUF + [sem] * _NEB + [sem] * _NBUF
        ),
    )
    def k(x_hbm, s_hbm, t_hbm, out_hbm, idx_v, *bufsem):
        xbs = bufsem[:_NBUF]
        ebs = bufsem[_NBUF:_NBUF + _NEB]
        lss = bufsem[_NBUF + _NEB:2 * _NBUF + _NEB]
        gss = bufsem[2 * _NBUF + _NEB:2 * _NBUF + 2 * _NEB]
        sss = bufsem[2 * _NBUF + 2 * _NEB:]
        wid = lax.axis_index("s") * 2 + lax.axis_index("c")
        base = wid * rows_w
        pltpu.sync_copy(s_hbm.at[pl.ds(base, rows_w)], idx_v)

        off = wid * _V

        def shift(j, cc):
            sl = pl.ds(j * _LANES, _LANES)
            idx_v[sl] = idx_v[sl] + off
            return cc

        lax.fori_loop(0, rows_w // _LANES, shift, 0, unroll=8)

        def issue_load(i, b):
            pltpu.async_copy(x_hbm.at[pl.ds(base + i * _CH, _CH)],
                             xbs[b], lss[b])

        def issue_gath(i, e):
            pltpu.async_copy(t_hbm.at[idx_v.at[pl.ds(i * _CH, _CH)]],
                             ebs[e], gss[e])

        for b in range(_NBUF):
            issue_load(b, b)
        for e in range(_NEB):
            issue_gath(e, e)

        def quad(i4, carry):
            for b in range(_NBUF):
                i = i4 * _NBUF + b
                e = b % _NEB
                pltpu.make_async_copy(x_hbm.at[pl.ds(0, _CH)],
                                      xbs[b], lss[b]).wait()
                pltpu.make_async_copy(x_hbm.at[pl.ds(0, _CH)],
                                      ebs[e], gss[e]).wait()
                xb, eb = xbs[b], ebs[e]

                def row(r, rc):
                    def col(c, cc):
                        sl = pl.ds(c * _LANES, _LANES)
                        plsc.addupdate(xb.at[r, sl], eb[r, sl])
                        return cc
                    lax.fori_loop(0, _D // _LANES, col, 0, unroll=8)
                    return rc

                lax.fori_loop(0, _CH, row, 0)

                @pl.when(i + _NEB < nch)
                def _():
                    issue_gath(i + _NEB, e)
                pltpu.async_copy(xb, out_hbm.at[pl.ds(base + i * _CH, _CH)],
                                 sss[b])
                jb = (b + _NBUF - 1) % _NBUF

                @pl.when(jnp.logical_and(i >= 1, i + _NBUF - 1 < nch))
                def _():
                    pltpu.make_async_copy(xbs[jb], out_hbm.at[pl.ds(0, _CH)],
                                          sss[jb]).wait()
                    issue_load(i + _NBUF - 1, jb)
            return carry

        lax.fori_loop(0, nch // _NBUF, quad, 0)
        for b in range(_NBUF):
            pltpu.make_async_copy(xbs[b], out_hbm.at[pl.ds(0, _CH)],
                                  sss[b]).wait()

    return k


def kernel(x, scores, score_embeddings):
    B, L, D = x.shape
    n = B * L
    xf = x.reshape(n, D)
    sf = scores.reshape(n).astype(jnp.int32)
    t_rep = jnp.tile(score_embeddings, (_NW, 1))
    out = _make_sc_kernel(n)(xf, sf, t_rep)
    return out.reshape(B, L, D)


# speedup vs baseline: 1.5379x; 1.0032x over previous
import functools

import jax
import jax.numpy as jnp
from jax import lax
from jax.experimental import pallas as pl
from jax.experimental.pallas import tpu as pltpu
from jax.experimental.pallas import tpu_sc as plsc

_D = 1024
_V = 11
_NW = 32
_CH = 16
_LANES = 16
_NBUF = 4
_NEB = 2
_BLK = 512
_N_SC = 8192


def _make_sc_kernel(n_rows):
    rows_w = n_rows // _NW
    nch = rows_w // _CH
    mesh = plsc.VectorSubcoreMesh(core_axis_name="c", subcore_axis_name="s")
    buf = pltpu.VMEM((_CH, _D), jnp.float32)
    sem = pltpu.SemaphoreType.DMA

    @functools.partial(
        pl.kernel,
        mesh=mesh,
        out_type=jax.ShapeDtypeStruct((n_rows, _D), jnp.float32),
        scratch_types=(
            [pltpu.VMEM((rows_w,), jnp.int32)]
            + [buf] * _NBUF + [buf] * _NEB
            + [sem] * _NBUF + [sem] * _NEB + [sem] * _NBUF
        ),
    )
    def k(x_hbm, s_hbm, t_hbm, out_hbm, idx_v, *bufsem):
        xbs = bufsem[:_NBUF]
        ebs = bufsem[_NBUF:_NBUF + _NEB]
        lss = bufsem[_NBUF + _NEB:2 * _NBUF + _NEB]
        gss = bufsem[2 * _NBUF + _NEB:2 * _NBUF + 2 * _NEB]
        sss = bufsem[2 * _NBUF + 2 * _NEB:]
        wid = lax.axis_index("s") * 2 + lax.axis_index("c")
        base = wid * rows_w
        pltpu.sync_copy(s_hbm.at[pl.ds(base, rows_w)], idx_v)

        off = wid * _V

        def shift(j, cc):
            sl = pl.ds(j * _LANES, _LANES)
            idx_v[sl] = idx_v[sl] + off
            return cc

        lax.fori_loop(0, rows_w // _LANES, shift, 0, unroll=8)

        def issue_load(i, b):
            pltpu.async_copy(x_hbm.at[pl.ds(base + i * _CH, _CH)],
                             xbs[b], lss[b])

        def issue_gath(i, e):
            pltpu.async_copy(t_hbm.at[idx_v.at[pl.ds(i * _CH, _CH)]],
                             ebs[e], gss[e])

        for b in range(_NBUF):
            issue_load(b, b)
        for e in range(_NEB):
            issue_gath(e, e)

        def quad(i4, carry):
            for b in range(_NBUF):
                i = i4 * _NBUF + b
                e = b % _NEB
                pltpu.make_async_copy(x_hbm.at[pl.ds(0, _CH)],
                                      xbs[b], lss[b]).wait()
                pltpu.make_async_copy(x_hbm.at[pl.ds(0, _CH)],
                                      ebs[e], gss[e]).wait()
                xb, eb = xbs[b], ebs[e]

                def row(r, rc):
                    def col(c, cc):
                        sl = pl.ds(c * _LANES, _LANES)
                        plsc.addupdate(xb.at[r, sl], eb[r, sl])
                        return cc
                    lax.fori_loop(0, _D // _LANES, col, 0, unroll=8)
                    return rc

                lax.fori_loop(0, _CH, row, 0)

                @pl.when(i + _NEB < nch)
                def _():
                    issue_gath(i + _NEB, e)
                pltpu.async_copy(xb, out_hbm.at[pl.ds(base + i * _CH, _CH)],
                                 sss[b])
                jb = (b + _NBUF - 1) % _NBUF

                @pl.when(jnp.logical_and(i >= 1, i + _NBUF - 1 < nch))
                def _():
                    pltpu.make_async_copy(xbs[jb], out_hbm.at[pl.ds(0, _CH)],
                                          sss[jb]).wait()
                    issue_load(i + _NBUF - 1, jb)
            return carry

        lax.fori_loop(0, nch // _NBUF, quad, 0)
        for b in range(_NBUF):
            pltpu.make_async_copy(xbs[b], out_hbm.at[pl.ds(0, _CH)],
                                  sss[b]).wait()

    return k


def _tc_body(s_ref, x_ref, t_ref, o_ref):
    s = s_ref[0, 0]
    oh = (s[:, None] == lax.broadcasted_iota(jnp.int32, (1, _V), 1)
          ).astype(jnp.float32)
    emb = jnp.dot(oh, t_ref[...], preferred_element_type=jnp.float32)
    o_ref[...] = x_ref[...] + emb


def _tc_call(xf, sf, table):
    n = xf.shape[0]
    return pl.pallas_call(
        _tc_body,
        grid=(n // _BLK,),
        in_specs=[
            pl.BlockSpec((1, 1, _BLK), lambda i: (i, 0, 0)),
            pl.BlockSpec((_BLK, _D), lambda i: (i, 0)),
            pl.BlockSpec((_V, _D), lambda i: (0, 0)),
        ],
        out_specs=pl.BlockSpec((_BLK, _D), lambda i: (i, 0)),
        out_shape=jax.ShapeDtypeStruct((n, _D), jnp.float32),
    )(sf.reshape(n // _BLK, 1, _BLK), xf, table)


def kernel(x, scores, score_embeddings):
    B, L, D = x.shape
    n = B * L
    n_tc = n - _N_SC
    xf = x.reshape(n, D)
    sf = scores.reshape(n).astype(jnp.int32)
    t_rep = jnp.tile(score_embeddings, (_NW, 1))
    out_sc = _make_sc_kernel(_N_SC)(xf[n_tc:], sf[n_tc:], t_rep)
    out_tc = _tc_call(xf[:n_tc], sf[:n_tc], score_embeddings)
    out = jnp.concatenate([out_tc, out_sc], axis=0)
    return out.reshape(B, L, D)
